# Initial kernel scaffold; baseline (speedup 1.0000x reference)
import jax
import jax.numpy as jnp
from jax.experimental import pallas as pl


def kernel(features, spconv_points_coors, spconv_points_coors_inv, scale_coors_inv, W_in, b_in, W1, b1, g1, be1, W2, b2, g2, be2, W3, b3, Wo1, bo1, Wo2, bo2):
    raise NotImplementedError("write your pallas kernel here")


# speedup vs baseline: 1.2618x; 1.2618x over previous
import jax
import jax.numpy as jnp
from jax.experimental import pallas as pl

_KN = 131072
_KB = 8192
_NB = 2000
_EPS = 1e-5


def _leaky(x):
    return jnp.where(x > 0, x, 0.1 * x)


def _stats_tile(t, m):
    s = jnp.sum(t * m, axis=0, keepdims=True)
    q = jnp.sum(t * t * m, axis=0, keepdims=True)
    mm = jnp.zeros((1, t.shape[1]), jnp.float32) + jnp.sum(m)
    return jnp.concatenate(
        [s, q, mm, jnp.zeros((5, t.shape[1]), jnp.float32)], axis=0)


def _acc(st_ref, tile):
    i = pl.program_id(0)

    @pl.when(i == 0)
    def _():
        st_ref[...] = tile

    @pl.when(i > 0)
    def _():
        st_ref[...] += tile


def _k_pass1(pin_ref, w_ref, b_ref, t_ref, st_ref):
    cnt = pin_ref[:, 16:17]
    pooled = pin_ref[:, :16] / jnp.maximum(cnt, 1.0)
    m = (cnt > 0).astype(jnp.float32)
    t = _leaky(jnp.dot(pooled, w_ref[...],
                       preferred_element_type=jnp.float32) + b_ref[0:1, :])
    t_ref[...] = t
    _acc(st_ref, _stats_tile(t, m))


def _k_pass2(t1_ref, pin_ref, ab_ref, w_ref, b_ref, t_ref, st_ref):
    m = (pin_ref[:, 16:17] > 0).astype(jnp.float32)
    x = t1_ref[...] * ab_ref[0:1, :] + ab_ref[1:2, :]
    t = _leaky(jnp.dot(x, w_ref[...],
                       preferred_element_type=jnp.float32) + b_ref[0:1, :])
    t_ref[...] = t
    _acc(st_ref, _stats_tile(t, m))


def _k_pass3(t2_ref, ab_ref, w_ref, b_ref, h_ref):
    x = t2_ref[...] * ab_ref[0:1, :] + ab_ref[1:2, :]
    h_ref[...] = _leaky(jnp.dot(x, w_ref[...],
                                preferred_element_type=jnp.float32)
                        + b_ref[0:1, :])


def _n_pass(f_ref, hg_ref, wi_ref, bi_ref, wa_ref, wb_ref, bo1_ref,
            wo2_ref, bo2_ref, y_ref):
    iden = _leaky(jnp.dot(f_ref[...], wi_ref[...],
                          preferred_element_type=jnp.float32) + bi_ref[0:1, :])
    t = _leaky(jnp.dot(iden, wa_ref[...], preferred_element_type=jnp.float32)
               + jnp.dot(hg_ref[...], wb_ref[...],
                         preferred_element_type=jnp.float32)
               + bo1_ref[0:1, :])
    y_ref[...] = (jnp.dot(t, wo2_ref[...], preferred_element_type=jnp.float32)
                  + bo2_ref[0:1, :])


def _row8(v):
    return jnp.concatenate(
        [v[None, :], jnp.zeros((7, v.shape[0]), jnp.float32)], axis=0)


def _ab8(a, c):
    return jnp.concatenate(
        [a[None, :], c[None, :], jnp.zeros((6, a.shape[0]), jnp.float32)],
        axis=0)


def _finalize(st, g, be):
    s, q, mm = st[0], st[1], st[2, 0]
    mean = s / mm
    var = q / mm - mean * mean
    a = g * jax.lax.rsqrt(var + _EPS)
    return _ab8(a, be - mean * a)


def kernel(features, spconv_points_coors, spconv_points_coors_inv,
           scale_coors_inv, W_in, b_in, W1, b1, g1, be1, W2, b2, g2, be2,
           W3, b3, Wo1, bo1, Wo2, bo2):
    n = features.shape[0]
    m_out = 60000
    batch = spconv_points_coors[:, 0]
    c = spconv_points_coors[:, 1:] // 2
    keyv = ((batch * 16 + c[:, 0]) * 16 + c[:, 1]) * 16 + c[:, 2]

    sums = jax.ops.segment_sum(features, keyv, num_segments=_KN)
    cnt = jax.ops.segment_sum(jnp.ones((n,), jnp.float32), keyv,
                              num_segments=_KN)
    pin = jnp.concatenate([sums, cnt[:, None]], axis=1)

    nkb = _KN // _KB
    kgrid = dict(grid=(nkb,))
    kmap = lambda i: (i, 0)
    fix = lambda i: (0, 0)

    t1, st1 = pl.pallas_call(
        _k_pass1,
        out_shape=(jax.ShapeDtypeStruct((_KN, 32), jnp.float32),
                   jax.ShapeDtypeStruct((8, 32), jnp.float32)),
        in_specs=[pl.BlockSpec((_KB, 17), kmap),
                  pl.BlockSpec((16, 32), fix),
                  pl.BlockSpec((8, 32), fix)],
        out_specs=(pl.BlockSpec((_KB, 32), kmap),
                   pl.BlockSpec((8, 32), fix)),
        **kgrid)(pin, W1, _row8(b1))
    ab1 = _finalize(st1, g1, be1)

    t2, st2 = pl.pallas_call(
        _k_pass2,
        out_shape=(jax.ShapeDtypeStruct((_KN, 32), jnp.float32),
                   jax.ShapeDtypeStruct((8, 32), jnp.float32)),
        in_specs=[pl.BlockSpec((_KB, 32), kmap),
                  pl.BlockSpec((_KB, 17), kmap),
                  pl.BlockSpec((8, 32), fix),
                  pl.BlockSpec((32, 32), fix),
                  pl.BlockSpec((8, 32), fix)],
        out_specs=(pl.BlockSpec((_KB, 32), kmap),
                   pl.BlockSpec((8, 32), fix)),
        **kgrid)(t1, pin, ab1, W2, _row8(b2))
    ab2 = _finalize(st2, g2, be2)

    h = pl.pallas_call(
        _k_pass3,
        out_shape=jax.ShapeDtypeStruct((_KN, 64), jnp.float32),
        in_specs=[pl.BlockSpec((_KB, 32), kmap),
                  pl.BlockSpec((8, 32), fix),
                  pl.BlockSpec((32, 64), fix),
                  pl.BlockSpec((8, 64), fix)],
        out_specs=pl.BlockSpec((_KB, 64), kmap),
        **kgrid)(t2, ab2, W3, _row8(b3))

    inv = spconv_points_coors_inv
    finv = jnp.take(features, inv, axis=0)
    hg = jnp.take(h, jnp.take(keyv, inv), axis=0)

    y = pl.pallas_call(
        _n_pass,
        out_shape=jax.ShapeDtypeStruct((n, 64), jnp.float32),
        in_specs=[pl.BlockSpec((_NB, 16), kmap),
                  pl.BlockSpec((_NB, 64), kmap),
                  pl.BlockSpec((16, 64), fix),
                  pl.BlockSpec((8, 64), fix),
                  pl.BlockSpec((64, 64), fix),
                  pl.BlockSpec((64, 64), fix),
                  pl.BlockSpec((8, 64), fix),
                  pl.BlockSpec((64, 64), fix),
                  pl.BlockSpec((8, 64), fix)],
        out_specs=pl.BlockSpec((_NB, 64), kmap),
        grid=(n // _NB,))(finv, hg, W_in, _row8(b_in), Wo1[:64], Wo1[64:],
                          _row8(bo1), Wo2, _row8(bo2))

    ssum = jax.ops.segment_sum(y, scale_coors_inv, num_segments=m_out)
    scnt = jax.ops.segment_sum(jnp.ones((n,), jnp.float32), scale_coors_inv,
                               num_segments=m_out)
    return ssum / jnp.maximum(scnt, 1.0)[:, None]


# speedup vs baseline: 1.4204x; 1.1257x over previous
import jax
import jax.numpy as jnp
from jax.experimental import pallas as pl

_KN = 131072
_KB = 8192
_NB = 2000
_EPS = 1e-5


def _leaky(x):
    return jnp.where(x > 0, x, 0.1 * x)


def _stats_tile(t, m):
    s = jnp.sum(t * m, axis=0, keepdims=True)
    q = jnp.sum(t * t * m, axis=0, keepdims=True)
    mm = jnp.zeros((1, t.shape[1]), jnp.float32) + jnp.sum(m)
    return jnp.concatenate(
        [s, q, mm, jnp.zeros((5, t.shape[1]), jnp.float32)], axis=0)


def _acc(st_ref, tile):
    i = pl.program_id(0)

    @pl.when(i == 0)
    def _():
        st_ref[...] = tile

    @pl.when(i > 0)
    def _():
        st_ref[...] += tile


def _k_pass1(pin_ref, w_ref, b_ref, t_ref, st_ref):
    cnt = pin_ref[:, 16:17]
    pooled = pin_ref[:, :16] / jnp.maximum(cnt, 1.0)
    m = (cnt > 0).astype(jnp.float32)
    t = _leaky(jnp.dot(pooled, w_ref[...],
                       preferred_element_type=jnp.float32) + b_ref[0:1, :])
    t_ref[...] = t
    _acc(st_ref, _stats_tile(t, m))


def _k_pass2(t1_ref, pin_ref, ab_ref, w_ref, b_ref, t_ref, st_ref):
    m = (pin_ref[:, 16:17] > 0).astype(jnp.float32)
    x = t1_ref[...] * ab_ref[0:1, :] + ab_ref[1:2, :]
    t = _leaky(jnp.dot(x, w_ref[...],
                       preferred_element_type=jnp.float32) + b_ref[0:1, :])
    t_ref[...] = t
    _acc(st_ref, _stats_tile(t, m))


def _k_pass3(t2_ref, ab_ref, w_ref, b_ref, h_ref):
    x = t2_ref[...] * ab_ref[0:1, :] + ab_ref[1:2, :]
    h_ref[...] = _leaky(jnp.dot(x, w_ref[...],
                                preferred_element_type=jnp.float32)
                        + b_ref[0:1, :])


def _n_pass(f_ref, hg_ref, wi_ref, bi_ref, wa_ref, wb_ref, bo1_ref,
            wo2_ref, bo2_ref, y_ref):
    iden = _leaky(jnp.dot(f_ref[...], wi_ref[...],
                          preferred_element_type=jnp.float32) + bi_ref[0:1, :])
    t = _leaky(jnp.dot(iden, wa_ref[...], preferred_element_type=jnp.float32)
               + jnp.dot(hg_ref[...], wb_ref[...],
                         preferred_element_type=jnp.float32)
               + bo1_ref[0:1, :])
    y_ref[...] = (jnp.dot(t, wo2_ref[...], preferred_element_type=jnp.float32)
                  + bo2_ref[0:1, :])


def _row8(v):
    return jnp.concatenate(
        [v[None, :], jnp.zeros((7, v.shape[0]), jnp.float32)], axis=0)


def _ab8(a, c):
    return jnp.concatenate(
        [a[None, :], c[None, :], jnp.zeros((6, a.shape[0]), jnp.float32)],
        axis=0)


def _finalize(st, g, be):
    s, q, mm = st[0], st[1], st[2, 0]
    mean = s / mm
    var = q / mm - mean * mean
    a = g * jax.lax.rsqrt(var + _EPS)
    return _ab8(a, be - mean * a)


def kernel(features, spconv_points_coors, spconv_points_coors_inv,
           scale_coors_inv, W_in, b_in, W1, b1, g1, be1, W2, b2, g2, be2,
           W3, b3, Wo1, bo1, Wo2, bo2):
    n = features.shape[0]
    m_out = 60000
    batch = spconv_points_coors[:, 0]
    c = spconv_points_coors[:, 1:] // 2
    keyv = ((batch * 16 + c[:, 0]) * 16 + c[:, 1]) * 16 + c[:, 2]

    faug = jnp.concatenate([features, jnp.ones((n, 1), jnp.float32)], axis=1)
    pin = jax.ops.segment_sum(faug, keyv, num_segments=_KN)

    nkb = _KN // _KB
    kgrid = dict(grid=(nkb,))
    kmap = lambda i: (i, 0)
    fix = lambda i: (0, 0)

    t1, st1 = pl.pallas_call(
        _k_pass1,
        out_shape=(jax.ShapeDtypeStruct((_KN, 32), jnp.float32),
                   jax.ShapeDtypeStruct((8, 32), jnp.float32)),
        in_specs=[pl.BlockSpec((_KB, 17), kmap),
                  pl.BlockSpec((16, 32), fix),
                  pl.BlockSpec((8, 32), fix)],
        out_specs=(pl.BlockSpec((_KB, 32), kmap),
                   pl.BlockSpec((8, 32), fix)),
        **kgrid)(pin, W1, _row8(b1))
    ab1 = _finalize(st1, g1, be1)

    t2, st2 = pl.pallas_call(
        _k_pass2,
        out_shape=(jax.ShapeDtypeStruct((_KN, 32), jnp.float32),
                   jax.ShapeDtypeStruct((8, 32), jnp.float32)),
        in_specs=[pl.BlockSpec((_KB, 32), kmap),
                  pl.BlockSpec((_KB, 17), kmap),
                  pl.BlockSpec((8, 32), fix),
                  pl.BlockSpec((32, 32), fix),
                  pl.BlockSpec((8, 32), fix)],
        out_specs=(pl.BlockSpec((_KB, 32), kmap),
                   pl.BlockSpec((8, 32), fix)),
        **kgrid)(t1, pin, ab1, W2, _row8(b2))
    ab2 = _finalize(st2, g2, be2)

    h = pl.pallas_call(
        _k_pass3,
        out_shape=jax.ShapeDtypeStruct((_KN, 64), jnp.float32),
        in_specs=[pl.BlockSpec((_KB, 32), kmap),
                  pl.BlockSpec((8, 32), fix),
                  pl.BlockSpec((32, 64), fix),
                  pl.BlockSpec((8, 64), fix)],
        out_specs=pl.BlockSpec((_KB, 64), kmap),
        **kgrid)(t2, ab2, W3, _row8(b3))

    inv = spconv_points_coors_inv
    finv = jnp.take(features, inv, axis=0)
    hg = jnp.take(h, jnp.take(keyv, inv), axis=0)

    y = pl.pallas_call(
        _n_pass,
        out_shape=jax.ShapeDtypeStruct((n, 64), jnp.float32),
        in_specs=[pl.BlockSpec((_NB, 16), kmap),
                  pl.BlockSpec((_NB, 64), kmap),
                  pl.BlockSpec((16, 64), fix),
                  pl.BlockSpec((8, 64), fix),
                  pl.BlockSpec((64, 64), fix),
                  pl.BlockSpec((64, 64), fix),
                  pl.BlockSpec((8, 64), fix),
                  pl.BlockSpec((64, 64), fix),
                  pl.BlockSpec((8, 64), fix)],
        out_specs=pl.BlockSpec((_NB, 64), kmap),
        grid=(n // _NB,))(finv, hg, W_in, _row8(b_in), Wo1[:64], Wo1[64:],
                          _row8(bo1), Wo2, _row8(bo2))

    yaug = jnp.concatenate([y, jnp.ones((n, 1), jnp.float32)], axis=1)
    sagg = jax.ops.segment_sum(yaug, scale_coors_inv, num_segments=m_out)
    return sagg[:, :64] / jnp.maximum(sagg[:, 64:], 1.0)


# speedup vs baseline: 1.4545x; 1.0240x over previous
import jax
import jax.numpy as jnp
from jax.experimental import pallas as pl

_KN = 131072
_KB = 8192
_NB = 2000
_EPS = 1e-5


def _leaky(x):
    return jnp.where(x > 0, x, 0.1 * x)


def _stats_tile(t, m):
    s = jnp.sum(t * m, axis=0, keepdims=True)
    q = jnp.sum(t * t * m, axis=0, keepdims=True)
    mm = jnp.zeros((1, t.shape[1]), jnp.float32) + jnp.sum(m)
    return jnp.concatenate(
        [s, q, mm, jnp.zeros((5, t.shape[1]), jnp.float32)], axis=0)


def _acc(st_ref, tile):
    i = pl.program_id(0)

    @pl.when(i == 0)
    def _():
        st_ref[...] = tile

    @pl.when(i > 0)
    def _():
        st_ref[...] += tile


def _k_pass1(pin_ref, w_ref, b_ref, t_ref, st_ref):
    cnt = pin_ref[:, 16:17]
    pooled = pin_ref[:, :16] / jnp.maximum(cnt, 1.0)
    m = (cnt > 0).astype(jnp.float32)
    t = _leaky(jnp.dot(pooled, w_ref[...],
                       preferred_element_type=jnp.float32) + b_ref[0:1, :])
    t_ref[...] = t
    _acc(st_ref, _stats_tile(t, m))


def _k_pass2(t1_ref, pin_ref, ab_ref, w_ref, b_ref, t_ref, st_ref):
    m = (pin_ref[:, 16:17] > 0).astype(jnp.float32)
    x = t1_ref[...] * ab_ref[0:1, :] + ab_ref[1:2, :]
    t = _leaky(jnp.dot(x, w_ref[...],
                       preferred_element_type=jnp.float32) + b_ref[0:1, :])
    t_ref[...] = t
    _acc(st_ref, _stats_tile(t, m))


def _k_pass3(t2_ref, ab_ref, w_ref, b_ref, h_ref):
    x = t2_ref[...] * ab_ref[0:1, :] + ab_ref[1:2, :]
    h_ref[...] = _leaky(jnp.dot(x, w_ref[...],
                                preferred_element_type=jnp.float32)
                        + b_ref[0:1, :])


def _n_pass(f_ref, hg_ref, wi_ref, bi_ref, wa_ref, wb_ref, bo1_ref,
            wo2_ref, bo2_ref, y_ref):
    iden = _leaky(jnp.dot(f_ref[...], wi_ref[...],
                          preferred_element_type=jnp.float32) + bi_ref[0:1, :])
    t = _leaky(jnp.dot(iden, wa_ref[...], preferred_element_type=jnp.float32)
               + jnp.dot(hg_ref[...], wb_ref[...],
                         preferred_element_type=jnp.float32)
               + bo1_ref[0:1, :])
    y = (jnp.dot(t, wo2_ref[...], preferred_element_type=jnp.float32)
         + bo2_ref[0:1, :])
    y_ref[...] = jnp.concatenate(
        [y, jnp.ones((y.shape[0], 1), jnp.float32)], axis=1)


def _row8(v):
    return jnp.concatenate(
        [v[None, :], jnp.zeros((7, v.shape[0]), jnp.float32)], axis=0)


def _ab8(a, c):
    return jnp.concatenate(
        [a[None, :], c[None, :], jnp.zeros((6, a.shape[0]), jnp.float32)],
        axis=0)


def _finalize(st, g, be):
    s, q, mm = st[0], st[1], st[2, 0]
    mean = s / mm
    var = q / mm - mean * mean
    a = g * jax.lax.rsqrt(var + _EPS)
    return _ab8(a, be - mean * a)


def kernel(features, spconv_points_coors, spconv_points_coors_inv,
           scale_coors_inv, W_in, b_in, W1, b1, g1, be1, W2, b2, g2, be2,
           W3, b3, Wo1, bo1, Wo2, bo2):
    n = features.shape[0]
    m_out = 60000
    batch = spconv_points_coors[:, 0]
    c = spconv_points_coors[:, 1:] // 2
    keyv = ((batch * 16 + c[:, 0]) * 16 + c[:, 1]) * 16 + c[:, 2]

    faug = jnp.concatenate([features, jnp.ones((n, 1), jnp.float32)], axis=1)
    pin = jax.ops.segment_sum(faug, keyv, num_segments=_KN)

    nkb = _KN // _KB
    kgrid = dict(grid=(nkb,))
    kmap = lambda i: (i, 0)
    fix = lambda i: (0, 0)

    t1, st1 = pl.pallas_call(
        _k_pass1,
        out_shape=(jax.ShapeDtypeStruct((_KN, 32), jnp.float32),
                   jax.ShapeDtypeStruct((8, 32), jnp.float32)),
        in_specs=[pl.BlockSpec((_KB, 17), kmap),
                  pl.BlockSpec((16, 32), fix),
                  pl.BlockSpec((8, 32), fix)],
        out_specs=(pl.BlockSpec((_KB, 32), kmap),
                   pl.BlockSpec((8, 32), fix)),
        **kgrid)(pin, W1, _row8(b1))
    ab1 = _finalize(st1, g1, be1)

    t2, st2 = pl.pallas_call(
        _k_pass2,
        out_shape=(jax.ShapeDtypeStruct((_KN, 32), jnp.float32),
                   jax.ShapeDtypeStruct((8, 32), jnp.float32)),
        in_specs=[pl.BlockSpec((_KB, 32), kmap),
                  pl.BlockSpec((_KB, 17), kmap),
                  pl.BlockSpec((8, 32), fix),
                  pl.BlockSpec((32, 32), fix),
                  pl.BlockSpec((8, 32), fix)],
        out_specs=(pl.BlockSpec((_KB, 32), kmap),
                   pl.BlockSpec((8, 32), fix)),
        **kgrid)(t1, pin, ab1, W2, _row8(b2))
    ab2 = _finalize(st2, g2, be2)

    h = pl.pallas_call(
        _k_pass3,
        out_shape=jax.ShapeDtypeStruct((_KN, 64), jnp.float32),
        in_specs=[pl.BlockSpec((_KB, 32), kmap),
                  pl.BlockSpec((8, 32), fix),
                  pl.BlockSpec((32, 64), fix),
                  pl.BlockSpec((8, 64), fix)],
        out_specs=pl.BlockSpec((_KB, 64), kmap),
        **kgrid)(t2, ab2, W3, _row8(b3))

    inv = spconv_points_coors_inv
    finv = jnp.take(features, inv, axis=0)
    hg = jnp.take(h, jnp.take(keyv, inv), axis=0)

    yaug = pl.pallas_call(
        _n_pass,
        out_shape=jax.ShapeDtypeStruct((n, 65), jnp.float32),
        in_specs=[pl.BlockSpec((_NB, 16), kmap),
                  pl.BlockSpec((_NB, 64), kmap),
                  pl.BlockSpec((16, 64), fix),
                  pl.BlockSpec((8, 64), fix),
                  pl.BlockSpec((64, 64), fix),
                  pl.BlockSpec((64, 64), fix),
                  pl.BlockSpec((8, 64), fix),
                  pl.BlockSpec((64, 64), fix),
                  pl.BlockSpec((8, 64), fix)],
        out_specs=pl.BlockSpec((_NB, 65), kmap),
        grid=(n // _NB,))(finv, hg, W_in, _row8(b_in), Wo1[:64], Wo1[64:],
                          _row8(bo1), Wo2, _row8(bo2))

    sagg = jax.ops.segment_sum(yaug, scale_coors_inv, num_segments=m_out)
    return sagg[:, :64] / jnp.maximum(sagg[:, 64:], 1.0)
